# baseline (speedup 1.0000x reference)
import jax
import jax.numpy as jnp
from jax import lax
from jax.experimental import pallas as pl
from jax.experimental.pallas import tpu as pltpu

_SCALE = 1.0 + 0.1
_N = 128
_F = 256
_CROWS = 32
_CPT = _N // _CROWS
_G = 128
_NG = 2
_GROWS = _G * _N


def _body(sz_ref, x_ref, o_ref, buf_a, buf_b, buf_c,
          sem_ra, sem_rb, sem_rc, sem_wa, sem_wb, sem_wc):
    bufs = (buf_a, buf_b, buf_c)
    rsems = (sem_ra, sem_rb, sem_rc)
    wsems = (sem_wa, sem_wb, sem_wc)

    def chunk_copies(g):
        buf, sem = bufs[g % 3], rsems[g % 3]
        for k in range(_G):
            t = g * _G + k
            s = sz_ref[t]
            for c in range(_CPT):
                yield s > c * _CROWS, pltpu.make_async_copy(
                    x_ref.at[pl.ds(t * _N + c * _CROWS, _CROWS)],
                    buf.at[pl.ds(k * _N + c * _CROWS, _CROWS)], sem)

    def read_group(g):
        for cond, copy in chunk_copies(g):
            @pl.when(cond)
            def _():
                copy.start()

    def wait_read_group(g):
        for cond, copy in chunk_copies(g):
            @pl.when(cond)
            def _():
                copy.wait()

    def compute(g):
        buf = bufs[g % 3]
        irows = lax.broadcasted_iota(jnp.int32, (_N, _F), 0)
        for k in range(_G):
            s = sz_ref[g * _G + k]
            scale = jnp.where(irows < s, _SCALE, 0.0).astype(jnp.float32)
            sl = pl.ds(k * _N, _N)
            buf[sl, :] = buf[sl, :] * scale

    def write_group(g):
        pltpu.make_async_copy(
            bufs[g % 3], o_ref.at[pl.ds(g * _GROWS, _GROWS)],
            wsems[g % 3]).start()

    def wait_write_group(g):
        pltpu.make_async_copy(
            bufs[g % 3], o_ref.at[pl.ds(g * _GROWS, _GROWS)],
            wsems[g % 3]).wait()

    read_group(0)
    for g in range(_NG):
        if g >= 2:
            wait_write_group(g - 2)
        if g + 1 < _NG:
            read_group(g + 1)
        wait_read_group(g)
        compute(g)
        write_group(g)
    wait_write_group(_NG - 2)
    wait_write_group(_NG - 1)


def kernel(input_tensor, batch_set_size):
    B, T, N, F = input_tensor.shape
    BT = B * T
    x = input_tensor.reshape(BT * N, F)
    sizes = batch_set_size.reshape(BT)

    out = pl.pallas_call(
        _body,
        in_specs=[
            pl.BlockSpec(memory_space=pltpu.MemorySpace.SMEM),
            pl.BlockSpec(memory_space=pltpu.MemorySpace.HBM),
        ],
        out_specs=pl.BlockSpec(memory_space=pltpu.MemorySpace.HBM),
        out_shape=jax.ShapeDtypeStruct((BT * N, F), input_tensor.dtype),
        scratch_shapes=[
            pltpu.VMEM((_GROWS, _F), jnp.float32),
            pltpu.VMEM((_GROWS, _F), jnp.float32),
            pltpu.VMEM((_GROWS, _F), jnp.float32),
            pltpu.SemaphoreType.DMA,
            pltpu.SemaphoreType.DMA,
            pltpu.SemaphoreType.DMA,
            pltpu.SemaphoreType.DMA,
            pltpu.SemaphoreType.DMA,
            pltpu.SemaphoreType.DMA,
        ],
    )(sizes, x)
    return out.reshape(B, T, N, F)


# speedup vs baseline: 1.0590x; 1.0590x over previous
import jax
import jax.numpy as jnp
from jax import lax
from jax.experimental import pallas as pl
from jax.experimental.pallas import tpu as pltpu

_SCALE = 1.0 + 0.1
_N = 128
_F = 256
_CROWS = 32
_CPT = _N // _CROWS
_G = 64
_NG = 4
_GROWS = _G * _N


def _body(sz_ref, x_ref, o_ref, buf_a, buf_b, buf_c,
          sem_ra, sem_rb, sem_rc, sem_wa, sem_wb, sem_wc):
    bufs = (buf_a, buf_b, buf_c)
    rsems = (sem_ra, sem_rb, sem_rc)
    wsems = (sem_wa, sem_wb, sem_wc)

    def chunk_copies(g):
        buf, sem = bufs[g % 3], rsems[g % 3]
        for k in range(_G):
            t = g * _G + k
            s = sz_ref[t]
            for c in range(_CPT):
                yield s > c * _CROWS, pltpu.make_async_copy(
                    x_ref.at[pl.ds(t * _N + c * _CROWS, _CROWS)],
                    buf.at[pl.ds(k * _N + c * _CROWS, _CROWS)], sem)

    def read_group(g):
        for cond, copy in chunk_copies(g):
            @pl.when(cond)
            def _():
                copy.start()

    def wait_read_group(g):
        for cond, copy in chunk_copies(g):
            @pl.when(cond)
            def _():
                copy.wait()

    def compute(g):
        buf = bufs[g % 3]
        irows = lax.broadcasted_iota(jnp.int32, (_N, _F), 0)
        for k in range(_G):
            s = sz_ref[g * _G + k]
            scale = jnp.where(irows < s, _SCALE, 0.0).astype(jnp.float32)
            sl = pl.ds(k * _N, _N)
            buf[sl, :] = buf[sl, :] * scale

    def write_group(g):
        pltpu.make_async_copy(
            bufs[g % 3], o_ref.at[pl.ds(g * _GROWS, _GROWS)],
            wsems[g % 3]).start()

    def wait_write_group(g):
        pltpu.make_async_copy(
            bufs[g % 3], o_ref.at[pl.ds(g * _GROWS, _GROWS)],
            wsems[g % 3]).wait()

    read_group(0)
    for g in range(_NG):
        if g >= 2:
            wait_write_group(g - 2)
        if g + 1 < _NG:
            read_group(g + 1)
        wait_read_group(g)
        compute(g)
        write_group(g)
    wait_write_group(_NG - 2)
    wait_write_group(_NG - 1)


def kernel(input_tensor, batch_set_size):
    B, T, N, F = input_tensor.shape
    BT = B * T
    x = input_tensor.reshape(BT * N, F)
    sizes = batch_set_size.reshape(BT)

    out = pl.pallas_call(
        _body,
        in_specs=[
            pl.BlockSpec(memory_space=pltpu.MemorySpace.SMEM),
            pl.BlockSpec(memory_space=pltpu.MemorySpace.HBM),
        ],
        out_specs=pl.BlockSpec(memory_space=pltpu.MemorySpace.HBM),
        out_shape=jax.ShapeDtypeStruct((BT * N, F), input_tensor.dtype),
        scratch_shapes=[
            pltpu.VMEM((_GROWS, _F), jnp.float32),
            pltpu.VMEM((_GROWS, _F), jnp.float32),
            pltpu.VMEM((_GROWS, _F), jnp.float32),
            pltpu.SemaphoreType.DMA,
            pltpu.SemaphoreType.DMA,
            pltpu.SemaphoreType.DMA,
            pltpu.SemaphoreType.DMA,
            pltpu.SemaphoreType.DMA,
            pltpu.SemaphoreType.DMA,
        ],
    )(sizes, x)
    return out.reshape(B, T, N, F)


# speedup vs baseline: 1.0742x; 1.0143x over previous
import jax
import jax.numpy as jnp
from jax import lax
from jax.experimental import pallas as pl
from jax.experimental.pallas import tpu as pltpu

_SCALE = 1.0 + 0.1
_N = 128
_F = 256
_CROWS = 16
_CPT = _N // _CROWS
_G = 64
_NG = 4
_GROWS = _G * _N


def _body(sz_ref, x_ref, o_ref, buf_a, buf_b, buf_c,
          sem_ra, sem_rb, sem_rc, sem_wa, sem_wb, sem_wc):
    bufs = (buf_a, buf_b, buf_c)
    rsems = (sem_ra, sem_rb, sem_rc)
    wsems = (sem_wa, sem_wb, sem_wc)

    def chunk_copies(g):
        buf, sem = bufs[g % 3], rsems[g % 3]
        for k in range(_G):
            t = g * _G + k
            s = sz_ref[t]
            for c in range(_CPT):
                yield s > c * _CROWS, pltpu.make_async_copy(
                    x_ref.at[pl.ds(t * _N + c * _CROWS, _CROWS)],
                    buf.at[pl.ds(k * _N + c * _CROWS, _CROWS)], sem)

    def read_group(g):
        for cond, copy in chunk_copies(g):
            @pl.when(cond)
            def _():
                copy.start()

    def wait_read_group(g):
        for cond, copy in chunk_copies(g):
            @pl.when(cond)
            def _():
                copy.wait()

    def compute(g):
        buf = bufs[g % 3]
        irows = lax.broadcasted_iota(jnp.int32, (_N, _F), 0)
        for k in range(_G):
            s = sz_ref[g * _G + k]
            scale = jnp.where(irows < s, _SCALE, 0.0).astype(jnp.float32)
            sl = pl.ds(k * _N, _N)
            buf[sl, :] = buf[sl, :] * scale

    def write_group(g):
        pltpu.make_async_copy(
            bufs[g % 3], o_ref.at[pl.ds(g * _GROWS, _GROWS)],
            wsems[g % 3]).start()

    def wait_write_group(g):
        pltpu.make_async_copy(
            bufs[g % 3], o_ref.at[pl.ds(g * _GROWS, _GROWS)],
            wsems[g % 3]).wait()

    read_group(0)
    for g in range(_NG):
        if g >= 2:
            wait_write_group(g - 2)
        if g + 1 < _NG:
            read_group(g + 1)
        wait_read_group(g)
        compute(g)
        write_group(g)
    wait_write_group(_NG - 2)
    wait_write_group(_NG - 1)


def kernel(input_tensor, batch_set_size):
    B, T, N, F = input_tensor.shape
    BT = B * T
    x = input_tensor.reshape(BT * N, F)
    sizes = batch_set_size.reshape(BT)

    out = pl.pallas_call(
        _body,
        in_specs=[
            pl.BlockSpec(memory_space=pltpu.MemorySpace.SMEM),
            pl.BlockSpec(memory_space=pltpu.MemorySpace.HBM),
        ],
        out_specs=pl.BlockSpec(memory_space=pltpu.MemorySpace.HBM),
        out_shape=jax.ShapeDtypeStruct((BT * N, F), input_tensor.dtype),
        scratch_shapes=[
            pltpu.VMEM((_GROWS, _F), jnp.float32),
            pltpu.VMEM((_GROWS, _F), jnp.float32),
            pltpu.VMEM((_GROWS, _F), jnp.float32),
            pltpu.SemaphoreType.DMA,
            pltpu.SemaphoreType.DMA,
            pltpu.SemaphoreType.DMA,
            pltpu.SemaphoreType.DMA,
            pltpu.SemaphoreType.DMA,
            pltpu.SemaphoreType.DMA,
        ],
    )(sizes, x)
    return out.reshape(B, T, N, F)
